# baseline (speedup 1.0000x reference)
import functools

import jax
import jax.numpy as jnp
from jax import lax
from jax.experimental import pallas as pl
from jax.experimental.pallas import tpu as pltpu
from jax.experimental.pallas import tpu_sc as plsc


def _ztab_body(x_ref, pos_ref, w1x_ref, w1p_ref, z_ref):
    x = x_ref[0]
    p = pos_ref[0]
    zx = lax.dot_general(x, w1x_ref[...], (((0,), (1,)), ((), ())),
                         preferred_element_type=jnp.float32)
    zp = lax.dot_general(p, w1p_ref[...], (((0,), (1,)), ((), ())),
                         preferred_element_type=jnp.float32)
    z_ref[0] = (zx + zp).astype(jnp.bfloat16)


def _make_ztab(B, C, N, H1):
    return pl.pallas_call(
        _ztab_body,
        grid=(B,),
        in_specs=[
            pl.BlockSpec((1, C, N), lambda b: (b, 0, 0)),
            pl.BlockSpec((1, 3, N), lambda b: (b, 0, 0)),
            pl.BlockSpec((H1, C), lambda b: (0, 0)),
            pl.BlockSpec((H1, 3), lambda b: (0, 0)),
        ],
        out_specs=pl.BlockSpec((1, N, H1), lambda b: (b, 0, 0)),
        out_shape=jax.ShapeDtypeStruct((B, N, H1), jnp.bfloat16),
    )


def _make_sc_gather(BR, H1, IDXW):
    info = plsc.get_sparse_core_info()
    NC, NS = info.num_cores, info.num_subcores
    NW = NC * NS
    rows_w = BR // NW
    SUB = 8
    CHUNK = SUB * IDXW
    nchunk = rows_w // CHUNK
    assert rows_w % CHUNK == 0
    PK = 128 // H1
    assert CHUNK % PK == 0 and BR % PK == 0

    mesh = plsc.VectorSubcoreMesh(core_axis_name="c", subcore_axis_name="s")

    @functools.partial(
        pl.kernel,
        mesh=mesh,
        out_type=jax.ShapeDtypeStruct((BR, H1), jnp.bfloat16),
        scratch_types=[
            pltpu.VMEM((SUB, IDXW), jnp.int32),
            pltpu.VMEM((CHUNK, H1), jnp.bfloat16),
            pltpu.SemaphoreType.DMA,
        ],
        compiler_params=pltpu.CompilerParams(use_tc_tiling_on_sc=False),
    )
    def k(tab_hbm, idx_hbm, out_hbm, idx_v, rows_v, sem):
        wid = lax.axis_index("s") * NC + lax.axis_index("c")
        base = wid * rows_w
        ibase = wid * (rows_w // IDXW)

        def chunk(g, carry):
            pltpu.sync_copy(idx_hbm.at[pl.ds(ibase + g * SUB, SUB)], idx_v)
            handles = []
            for j in range(SUB):
                handles.append(
                    pltpu.async_copy(tab_hbm.at[idx_v.at[j]],
                                     rows_v.at[pl.ds(j * IDXW, IDXW)], sem))
            for h in handles:
                h.wait()
            pltpu.sync_copy(rows_v,
                            out_hbm.at[pl.ds(base + g * CHUNK, CHUNK)])
            return carry

        lax.fori_loop(0, nchunk, chunk, 0)

    return k


def _head_body(K, MB, PK, g_ref, sup_ref, w1p_ref, b1_ref, w2b_ref, b2_ref,
               o_ref):
    KP = K // PK
    H1 = w1p_ref.shape[0]
    t = lax.dot_general(sup_ref[0], w1p_ref[...], (((1,), (1,)), ((), ())),
                        preferred_element_type=jnp.float32) - b1_ref[...]
    t4 = jnp.concatenate([t] * PK, axis=1)
    g3 = g_ref[0].astype(jnp.float32).reshape(MB, KP, PK * H1)
    r = jnp.maximum(g3 - t4[:, None, :], 0.0).reshape(MB * KP, PK * H1)
    h = lax.dot_general(r, w2b_ref[...], (((1,), (0,)), ((), ())),
                        preferred_element_type=jnp.float32)
    OUT = o_ref.shape[2]
    hm = jnp.max(h.reshape(MB, KP, PK * OUT), axis=1)
    o = hm[:, :OUT]
    for q in range(1, PK):
        o = jnp.maximum(o, hm[:, q * OUT:(q + 1) * OUT])
    o_ref[0] = o + b2_ref[...]


def _make_head(B, M, K, H1, OUT, MB, PK):
    nmb = M // MB
    KP = K // PK
    return pl.pallas_call(
        functools.partial(_head_body, K, MB, PK),
        grid=(B, nmb),
        in_specs=[
            pl.BlockSpec((1, MB * KP, PK * H1), lambda b, i: (b, i, 0)),
            pl.BlockSpec((1, MB, 3), lambda b, i: (b, i, 0)),
            pl.BlockSpec((H1, 3), lambda b, i: (0, 0)),
            pl.BlockSpec((1, H1), lambda b, i: (0, 0)),
            pl.BlockSpec((PK * H1, PK * OUT), lambda b, i: (0, 0)),
            pl.BlockSpec((1, OUT), lambda b, i: (0, 0)),
        ],
        out_specs=pl.BlockSpec((1, MB, OUT), lambda b, i: (b, i, 0)),
        out_shape=jax.ShapeDtypeStruct((B, M, OUT), jnp.float32),
        compiler_params=pltpu.CompilerParams(
            dimension_semantics=("parallel", "parallel")),
    )


def kernel(x, pos, support_points, indices, W1, b1, W2, b2):
    B, C, N = x.shape
    _, M, K = indices.shape
    H1 = W1.shape[0]
    OUT = W2.shape[0]
    W1x = W1[:, :C]
    W1p = W1[:, C:]

    z = _make_ztab(B, C, N, H1)(x, pos, W1x, W1p)

    BR = B * M * K
    IDXW = 100
    PK = 128 // H1
    MB = 400
    KP = K // PK
    idxf = (indices.reshape(B, M * K)
            + (jnp.arange(B, dtype=jnp.int32) * N)[:, None]
            ).reshape(BR // IDXW, IDXW)
    g = _make_sc_gather(BR, H1, IDXW)(z.reshape(B * N, H1), idxf)
    g4 = g.reshape(B, M * K // PK, PK * H1)

    supT = support_points.transpose(0, 2, 1)
    W2blk = jnp.kron(jnp.eye(PK, dtype=W2.dtype), W2.T)
    out = _make_head(B, M, K, H1, OUT, MB=MB, PK=PK)(
        g4, supT, W1p,
        b1.reshape(1, H1), W2blk, b2.reshape(1, OUT))
    return out.transpose(0, 2, 1)


# speedup vs baseline: 1.7608x; 1.7608x over previous
import functools

import jax
import jax.numpy as jnp
from jax import lax
from jax.experimental import pallas as pl
from jax.experimental.pallas import tpu as pltpu
from jax.experimental.pallas import tpu_sc as plsc


def _ztab_body(x_ref, pos_ref, w1x_ref, w1p_ref, z_ref):
    x = x_ref[0]
    p = pos_ref[0]
    zx = lax.dot_general(x, w1x_ref[...], (((0,), (1,)), ((), ())),
                         preferred_element_type=jnp.float32)
    zp = lax.dot_general(p, w1p_ref[...], (((0,), (1,)), ((), ())),
                         preferred_element_type=jnp.float32)
    z_ref[0] = zx + zp


def _make_ztab(B, C, N, H1):
    return pl.pallas_call(
        _ztab_body,
        grid=(B,),
        in_specs=[
            pl.BlockSpec((1, C, N), lambda b: (b, 0, 0)),
            pl.BlockSpec((1, 3, N), lambda b: (b, 0, 0)),
            pl.BlockSpec((H1, C), lambda b: (0, 0)),
            pl.BlockSpec((H1, 3), lambda b: (0, 0)),
        ],
        out_specs=pl.BlockSpec((1, N, H1), lambda b: (b, 0, 0)),
        out_shape=jax.ShapeDtypeStruct((B, N, H1), jnp.float32),
    )


def _make_sc_gather(BR, H1, IDXW):
    info = plsc.get_sparse_core_info()
    NC, NS = info.num_cores, info.num_subcores
    NW = NC * NS
    rows_w = BR // NW
    SUB = 10
    CHUNK = SUB * IDXW
    nchunk = rows_w // CHUNK
    assert rows_w % CHUNK == 0
    PK = 128 // H1
    assert CHUNK % PK == 0 and BR % PK == 0

    mesh = plsc.VectorSubcoreMesh(core_axis_name="c", subcore_axis_name="s")

    @functools.partial(
        pl.kernel,
        mesh=mesh,
        out_type=jax.ShapeDtypeStruct((BR, H1), jnp.float32),
        scratch_types=[
            pltpu.VMEM((SUB, IDXW), jnp.int32),
            pltpu.VMEM((CHUNK, H1), jnp.float32),
            pltpu.SemaphoreType.DMA,
        ],
        compiler_params=pltpu.CompilerParams(use_tc_tiling_on_sc=False),
    )
    def k(tab_hbm, idx_hbm, out_hbm, idx_v, rows_v, sem):
        wid = lax.axis_index("s") * NC + lax.axis_index("c")
        base = wid * rows_w
        ibase = wid * (rows_w // IDXW)

        def chunk(g, carry):
            pltpu.sync_copy(idx_hbm.at[pl.ds(ibase + g * SUB, SUB)], idx_v)
            handles = []
            for j in range(SUB):
                handles.append(
                    pltpu.async_copy(tab_hbm.at[idx_v.at[j]],
                                     rows_v.at[pl.ds(j * IDXW, IDXW)], sem))
            for h in handles:
                h.wait()
            pltpu.sync_copy(rows_v,
                            out_hbm.at[pl.ds(base + g * CHUNK, CHUNK)])
            return carry

        lax.fori_loop(0, nchunk, chunk, 0)

    return k


def _head_body(K, MB, PK, g_ref, sup_ref, w1p_ref, b1_ref, w2b_ref, b2_ref,
               o_ref):
    KP = K // PK
    H1 = w1p_ref.shape[0]
    t = lax.dot_general(sup_ref[0], w1p_ref[...], (((1,), (1,)), ((), ())),
                        preferred_element_type=jnp.float32) - b1_ref[...]
    t4 = jnp.concatenate([t] * PK, axis=1)
    g3 = g_ref[0].reshape(MB, KP, PK * H1)
    r = jnp.maximum(g3 - t4[:, None, :], 0.0).reshape(MB * KP, PK * H1)
    h = lax.dot_general(r, w2b_ref[...], (((1,), (0,)), ((), ())),
                        preferred_element_type=jnp.float32)
    OUT = o_ref.shape[2]
    hm = jnp.max(h.reshape(MB, KP, PK * OUT), axis=1)
    o = hm[:, :OUT]
    for q in range(1, PK):
        o = jnp.maximum(o, hm[:, q * OUT:(q + 1) * OUT])
    o_ref[0] = o + b2_ref[...]


def _make_head(B, M, K, H1, OUT, MB, PK):
    nmb = M // MB
    KP = K // PK
    return pl.pallas_call(
        functools.partial(_head_body, K, MB, PK),
        grid=(B, nmb),
        in_specs=[
            pl.BlockSpec((1, MB * KP, PK * H1), lambda b, i: (b, i, 0)),
            pl.BlockSpec((1, MB, 3), lambda b, i: (b, i, 0)),
            pl.BlockSpec((H1, 3), lambda b, i: (0, 0)),
            pl.BlockSpec((1, H1), lambda b, i: (0, 0)),
            pl.BlockSpec((PK * H1, PK * OUT), lambda b, i: (0, 0)),
            pl.BlockSpec((1, OUT), lambda b, i: (0, 0)),
        ],
        out_specs=pl.BlockSpec((1, MB, OUT), lambda b, i: (b, i, 0)),
        out_shape=jax.ShapeDtypeStruct((B, M, OUT), jnp.float32),
        compiler_params=pltpu.CompilerParams(
            dimension_semantics=("parallel", "parallel")),
    )


def kernel(x, pos, support_points, indices, W1, b1, W2, b2):
    B, C, N = x.shape
    _, M, K = indices.shape
    H1 = W1.shape[0]
    OUT = W2.shape[0]
    W1x = W1[:, :C]
    W1p = W1[:, C:]

    z = _make_ztab(B, C, N, H1)(x, pos, W1x, W1p)

    BRb = M * K
    IDXW = 100
    PK = 128 // H1
    MB = 400
    KP = K // PK
    supT = support_points.transpose(0, 2, 1)
    W2blk = jnp.kron(jnp.eye(PK, dtype=W2.dtype), W2.T)
    b1r = b1.reshape(1, H1)
    b2r = b2.reshape(1, OUT)

    gather = _make_sc_gather(BRb, H1, IDXW)
    head = _make_head(1, M, K, H1, OUT, MB=MB, PK=PK)
    outs = []
    for b in range(B):
        idxf = indices[b].reshape(BRb // IDXW, IDXW)
        g = gather(z[b], idxf)
        g4 = g.reshape(1, M * K // PK, PK * H1)
        outs.append(head(g4, supT[b:b + 1], W1p, b1r, W2blk, b2r))
    out = jnp.concatenate(outs, axis=0)
    return out.transpose(0, 2, 1)


# speedup vs baseline: 1.8053x; 1.0253x over previous
import functools

import jax
import jax.numpy as jnp
from jax import lax
from jax.experimental import pallas as pl
from jax.experimental.pallas import tpu as pltpu
from jax.experimental.pallas import tpu_sc as plsc


def _ztab_body(x_ref, pos_ref, w1x_ref, w1p_ref, z_ref):
    x = x_ref[0]
    p = pos_ref[0]
    zx = lax.dot_general(x, w1x_ref[...], (((0,), (1,)), ((), ())),
                         preferred_element_type=jnp.float32)
    zp = lax.dot_general(p, w1p_ref[...], (((0,), (1,)), ((), ())),
                         preferred_element_type=jnp.float32)
    z_ref[0] = zx + zp


def _make_ztab(B, C, N, H1):
    return pl.pallas_call(
        _ztab_body,
        grid=(B,),
        in_specs=[
            pl.BlockSpec((1, C, N), lambda b: (b, 0, 0)),
            pl.BlockSpec((1, 3, N), lambda b: (b, 0, 0)),
            pl.BlockSpec((H1, C), lambda b: (0, 0)),
            pl.BlockSpec((H1, 3), lambda b: (0, 0)),
        ],
        out_specs=pl.BlockSpec((1, N, H1), lambda b: (b, 0, 0)),
        out_shape=jax.ShapeDtypeStruct((B, N, H1), jnp.float32),
    )


def _make_sc_gather(BR, H1, IDXW):
    info = plsc.get_sparse_core_info()
    NC, NS = info.num_cores, info.num_subcores
    NW = NC * NS
    rows_w = BR // NW
    SUB = 1
    CHUNK = SUB * IDXW
    nchunk = rows_w // CHUNK
    assert rows_w % CHUNK == 0
    PK = 128 // H1
    assert CHUNK % PK == 0 and BR % PK == 0

    mesh = plsc.VectorSubcoreMesh(core_axis_name="c", subcore_axis_name="s")

    @functools.partial(
        pl.kernel,
        mesh=mesh,
        out_type=jax.ShapeDtypeStruct((BR, H1), jnp.float32),
        scratch_types=[
            pltpu.VMEM((SUB, IDXW), jnp.int32),
            pltpu.VMEM((CHUNK, H1), jnp.float32),
            pltpu.SemaphoreType.DMA,
        ],
        compiler_params=pltpu.CompilerParams(use_tc_tiling_on_sc=False),
    )
    def k(tab_hbm, idx_hbm, out_hbm, idx_v, rows_v, sem):
        wid = lax.axis_index("s") * NC + lax.axis_index("c")
        base = wid * rows_w
        ibase = wid * (rows_w // IDXW)

        def chunk(g, carry):
            pltpu.sync_copy(idx_hbm.at[pl.ds(ibase + g * SUB, SUB)], idx_v)
            handles = []
            for j in range(SUB):
                handles.append(
                    pltpu.async_copy(tab_hbm.at[idx_v.at[j]],
                                     rows_v.at[pl.ds(j * IDXW, IDXW)], sem))
            for h in handles:
                h.wait()
            pltpu.sync_copy(rows_v,
                            out_hbm.at[pl.ds(base + g * CHUNK, CHUNK)])
            return carry

        lax.fori_loop(0, nchunk, chunk, 0)

    return k


def _head_body(K, MB, PK, g_ref, sup_ref, w1p_ref, b1_ref, w2b_ref, b2_ref,
               o_ref):
    KP = K // PK
    H1 = w1p_ref.shape[0]
    t = lax.dot_general(sup_ref[0], w1p_ref[...], (((1,), (1,)), ((), ())),
                        preferred_element_type=jnp.float32) - b1_ref[...]
    t4 = jnp.concatenate([t] * PK, axis=1)
    g3 = g_ref[0].reshape(MB, KP, PK * H1)
    r = jnp.maximum(g3 - t4[:, None, :], 0.0).reshape(MB * KP, PK * H1)
    h = lax.dot_general(r, w2b_ref[...], (((1,), (0,)), ((), ())),
                        preferred_element_type=jnp.float32)
    OUT = o_ref.shape[2]
    hm = jnp.max(h.reshape(MB, KP, PK * OUT), axis=1)
    o = hm[:, :OUT]
    for q in range(1, PK):
        o = jnp.maximum(o, hm[:, q * OUT:(q + 1) * OUT])
    o_ref[0] = o + b2_ref[...]


def _make_head(B, M, K, H1, OUT, MB, PK):
    nmb = M // MB
    KP = K // PK
    return pl.pallas_call(
        functools.partial(_head_body, K, MB, PK),
        grid=(B, nmb),
        in_specs=[
            pl.BlockSpec((1, MB * KP, PK * H1), lambda b, i: (b, i, 0)),
            pl.BlockSpec((1, MB, 3), lambda b, i: (b, i, 0)),
            pl.BlockSpec((H1, 3), lambda b, i: (0, 0)),
            pl.BlockSpec((1, H1), lambda b, i: (0, 0)),
            pl.BlockSpec((PK * H1, PK * OUT), lambda b, i: (0, 0)),
            pl.BlockSpec((1, OUT), lambda b, i: (0, 0)),
        ],
        out_specs=pl.BlockSpec((1, MB, OUT), lambda b, i: (b, i, 0)),
        out_shape=jax.ShapeDtypeStruct((B, M, OUT), jnp.float32),
        compiler_params=pltpu.CompilerParams(
            dimension_semantics=("parallel", "parallel")),
    )


def kernel(x, pos, support_points, indices, W1, b1, W2, b2):
    B, C, N = x.shape
    _, M, K = indices.shape
    H1 = W1.shape[0]
    OUT = W2.shape[0]
    W1x = W1[:, :C]
    W1p = W1[:, C:]

    z = _make_ztab(B, C, N, H1)(x, pos, W1x, W1p)

    BRb = M * K
    IDXW = 1000
    PK = 128 // H1
    MB = 400
    KP = K // PK
    supT = support_points.transpose(0, 2, 1)
    W2blk = jnp.kron(jnp.eye(PK, dtype=W2.dtype), W2.T)
    b1r = b1.reshape(1, H1)
    b2r = b2.reshape(1, OUT)

    gather = _make_sc_gather(BRb, H1, IDXW)
    head = _make_head(1, M, K, H1, OUT, MB=MB, PK=PK)
    outs = []
    for b in range(B):
        idxf = indices[b].reshape(BRb // IDXW, IDXW)
        g = gather(z[b], idxf)
        g4 = g.reshape(1, M * K // PK, PK * H1)
        outs.append(head(g4, supT[b:b + 1], W1p, b1r, W2blk, b2r))
    out = jnp.concatenate(outs, axis=0)
    return out.transpose(0, 2, 1)


# speedup vs baseline: 1.8091x; 1.0021x over previous
import functools

import jax
import jax.numpy as jnp
from jax import lax
from jax.experimental import pallas as pl
from jax.experimental.pallas import tpu as pltpu
from jax.experimental.pallas import tpu_sc as plsc


def _ztab_body(x_ref, pos_ref, w1x_ref, w1p_ref, z_ref):
    x = x_ref[0]
    p = pos_ref[0]
    zx = lax.dot_general(x, w1x_ref[...], (((0,), (1,)), ((), ())),
                         preferred_element_type=jnp.float32)
    zp = lax.dot_general(p, w1p_ref[...], (((0,), (1,)), ((), ())),
                         preferred_element_type=jnp.float32)
    z_ref[0] = zx + zp


def _make_ztab(B, C, N, H1):
    return pl.pallas_call(
        _ztab_body,
        grid=(B,),
        in_specs=[
            pl.BlockSpec((1, C, N), lambda b: (b, 0, 0)),
            pl.BlockSpec((1, 3, N), lambda b: (b, 0, 0)),
            pl.BlockSpec((H1, C), lambda b: (0, 0)),
            pl.BlockSpec((H1, 3), lambda b: (0, 0)),
        ],
        out_specs=pl.BlockSpec((1, N, H1), lambda b: (b, 0, 0)),
        out_shape=jax.ShapeDtypeStruct((B, N, H1), jnp.float32),
    )


def _make_sc_gather(BR, H1, IDXW):
    info = plsc.get_sparse_core_info()
    NC, NS = info.num_cores, info.num_subcores
    NW = NC * NS
    rows_w = BR // NW
    SUB = 1
    CHUNK = SUB * IDXW
    nchunk = rows_w // CHUNK
    assert rows_w % CHUNK == 0
    PK = 128 // H1
    assert CHUNK % PK == 0 and BR % PK == 0

    assert nchunk % 2 == 0
    nhalf = nchunk // 2
    mesh = plsc.VectorSubcoreMesh(core_axis_name="c", subcore_axis_name="s")

    @functools.partial(
        pl.kernel,
        mesh=mesh,
        out_type=jax.ShapeDtypeStruct((BR, H1), jnp.float32),
        scratch_types=[
            pltpu.VMEM((SUB, IDXW), jnp.int32),
            pltpu.VMEM((SUB, IDXW), jnp.int32),
            pltpu.VMEM((CHUNK, H1), jnp.float32),
            pltpu.VMEM((CHUNK, H1), jnp.float32),
            pltpu.SemaphoreType.DMA,
            pltpu.SemaphoreType.DMA,
            pltpu.SemaphoreType.DMA,
            pltpu.SemaphoreType.DMA,
        ],
        compiler_params=pltpu.CompilerParams(use_tc_tiling_on_sc=False),
    )
    def k(tab_hbm, idx_hbm, out_hbm, i0, i1, r0, r1, sg0, sg1, ss0, ss1):
        wid = lax.axis_index("s") * NC + lax.axis_index("c")
        base = wid * rows_w
        ibase = wid * (rows_w // IDXW)

        def idxload(buf, c):
            pltpu.sync_copy(idx_hbm.at[pl.ds(ibase + c * SUB, SUB)], buf)

        def fire_g(ib, rb, sem):
            for j in range(SUB):
                pltpu.async_copy(tab_hbm.at[ib.at[j]],
                                 rb.at[pl.ds(j * IDXW, IDXW)], sem)

        def wait_g(ib, rb, sem):
            for j in range(SUB):
                pltpu.make_async_copy(tab_hbm.at[ib.at[j]],
                                      rb.at[pl.ds(j * IDXW, IDXW)], sem).wait()

        def fire_s(rb, c, sem):
            pltpu.async_copy(rb, out_hbm.at[pl.ds(base + c * CHUNK, CHUNK)], sem)

        def wait_s(rb, sem):
            pltpu.make_async_copy(rb, out_hbm.at[pl.ds(base, CHUNK)], sem).wait()

        idxload(i0, 0)
        fire_g(i0, r0, sg0)

        def body(i, carry):
            c0 = 2 * i
            c1 = c0 + 1
            idxload(i1, c1)

            @pl.when(i > 0)
            def _():
                wait_s(r1, ss1)

            fire_g(i1, r1, sg1)
            wait_g(i0, r0, sg0)
            fire_s(r0, c0, ss0)

            @pl.when(i < nhalf - 1)
            def _():
                idxload(i0, c0 + 2)
                wait_s(r0, ss0)
                fire_g(i0, r0, sg0)

            wait_g(i1, r1, sg1)
            fire_s(r1, c1, ss1)
            return carry

        lax.fori_loop(0, nhalf, body, 0)
        wait_s(r0, ss0)
        wait_s(r1, ss1)

    return k


def _head_body(K, MB, PK, g_ref, sup_ref, w1p_ref, b1_ref, w2b_ref, b2_ref,
               o_ref):
    KP = K // PK
    H1 = w1p_ref.shape[0]
    t = lax.dot_general(sup_ref[0], w1p_ref[...], (((1,), (1,)), ((), ())),
                        preferred_element_type=jnp.float32) - b1_ref[...]
    t4 = jnp.concatenate([t] * PK, axis=1)
    g3 = g_ref[0].reshape(MB, KP, PK * H1)
    r = jnp.maximum(g3 - t4[:, None, :], 0.0).reshape(MB * KP, PK * H1)
    h = lax.dot_general(r, w2b_ref[...], (((1,), (0,)), ((), ())),
                        preferred_element_type=jnp.float32)
    OUT = o_ref.shape[2]
    hm = jnp.max(h.reshape(MB, KP, PK * OUT), axis=1)
    o = hm[:, :OUT]
    for q in range(1, PK):
        o = jnp.maximum(o, hm[:, q * OUT:(q + 1) * OUT])
    o_ref[0] = o + b2_ref[...]


def _make_head(B, M, K, H1, OUT, MB, PK):
    nmb = M // MB
    KP = K // PK
    return pl.pallas_call(
        functools.partial(_head_body, K, MB, PK),
        grid=(B, nmb),
        in_specs=[
            pl.BlockSpec((1, MB * KP, PK * H1), lambda b, i: (b, i, 0)),
            pl.BlockSpec((1, MB, 3), lambda b, i: (b, i, 0)),
            pl.BlockSpec((H1, 3), lambda b, i: (0, 0)),
            pl.BlockSpec((1, H1), lambda b, i: (0, 0)),
            pl.BlockSpec((PK * H1, PK * OUT), lambda b, i: (0, 0)),
            pl.BlockSpec((1, OUT), lambda b, i: (0, 0)),
        ],
        out_specs=pl.BlockSpec((1, MB, OUT), lambda b, i: (b, i, 0)),
        out_shape=jax.ShapeDtypeStruct((B, M, OUT), jnp.float32),
        compiler_params=pltpu.CompilerParams(
            dimension_semantics=("parallel", "parallel")),
    )


def kernel(x, pos, support_points, indices, W1, b1, W2, b2):
    B, C, N = x.shape
    _, M, K = indices.shape
    H1 = W1.shape[0]
    OUT = W2.shape[0]
    W1x = W1[:, :C]
    W1p = W1[:, C:]

    z = _make_ztab(B, C, N, H1)(x, pos, W1x, W1p)

    BRb = M * K
    IDXW = 1000
    PK = 128 // H1
    MB = 400
    KP = K // PK
    supT = support_points.transpose(0, 2, 1)
    W2blk = jnp.kron(jnp.eye(PK, dtype=W2.dtype), W2.T)
    b1r = b1.reshape(1, H1)
    b2r = b2.reshape(1, OUT)

    gather = _make_sc_gather(BRb, H1, IDXW)
    head = _make_head(1, M, K, H1, OUT, MB=MB, PK=PK)
    outs = []
    for b in range(B):
        idxf = indices[b].reshape(BRb // IDXW, IDXW)
        g = gather(z[b], idxf)
        g4 = g.reshape(1, M * K // PK, PK * H1)
        outs.append(head(g4, supT[b:b + 1], W1p, b1r, W2blk, b2r))
    out = jnp.concatenate(outs, axis=0)
    return out.transpose(0, 2, 1)


# speedup vs baseline: 1.9022x; 1.0514x over previous
import functools

import jax
import jax.numpy as jnp
from jax import lax
from jax.experimental import pallas as pl
from jax.experimental.pallas import tpu as pltpu
from jax.experimental.pallas import tpu_sc as plsc


def _ztab_body(x_ref, pos_ref, w1x_ref, w1p_ref, z_ref):
    x = x_ref[0]
    p = pos_ref[0]
    zx = lax.dot_general(x, w1x_ref[...], (((0,), (1,)), ((), ())),
                         preferred_element_type=jnp.float32)
    zp = lax.dot_general(p, w1p_ref[...], (((0,), (1,)), ((), ())),
                         preferred_element_type=jnp.float32)
    z_ref[0] = zx + zp


def _make_ztab(B, C, N, H1):
    return pl.pallas_call(
        _ztab_body,
        grid=(B,),
        in_specs=[
            pl.BlockSpec((1, C, N), lambda b: (b, 0, 0)),
            pl.BlockSpec((1, 3, N), lambda b: (b, 0, 0)),
            pl.BlockSpec((H1, C), lambda b: (0, 0)),
            pl.BlockSpec((H1, 3), lambda b: (0, 0)),
        ],
        out_specs=pl.BlockSpec((1, N, H1), lambda b: (b, 0, 0)),
        out_shape=jax.ShapeDtypeStruct((B, N, H1), jnp.float32),
    )


def _make_sc_gather(BR, H1, IDXW):
    info = plsc.get_sparse_core_info()
    NC, NS = info.num_cores, info.num_subcores
    NW = NC * NS
    rows_w = BR // NW
    SUB = 1
    CHUNK = SUB * IDXW
    nchunk = rows_w // CHUNK
    assert rows_w % CHUNK == 0
    PK = 128 // H1
    assert CHUNK % PK == 0 and BR % PK == 0

    assert nchunk % 2 == 0
    nhalf = nchunk // 2
    mesh = plsc.VectorSubcoreMesh(core_axis_name="c", subcore_axis_name="s")

    @functools.partial(
        pl.kernel,
        mesh=mesh,
        out_type=jax.ShapeDtypeStruct((BR, H1), jnp.float32),
        scratch_types=[
            pltpu.VMEM((SUB, IDXW), jnp.int32),
            pltpu.VMEM((SUB, IDXW), jnp.int32),
            pltpu.VMEM((CHUNK, H1), jnp.float32),
            pltpu.VMEM((CHUNK, H1), jnp.float32),
            pltpu.SemaphoreType.DMA,
            pltpu.SemaphoreType.DMA,
            pltpu.SemaphoreType.DMA,
            pltpu.SemaphoreType.DMA,
        ],
        compiler_params=pltpu.CompilerParams(use_tc_tiling_on_sc=False),
    )
    def k(tab_hbm, idx_hbm, out_hbm, i0, i1, r0, r1, sg0, sg1, ss0, ss1):
        wid = lax.axis_index("s") * NC + lax.axis_index("c")
        base = wid * rows_w
        ibase = wid * (rows_w // IDXW)

        def idxload(buf, c):
            pltpu.sync_copy(idx_hbm.at[pl.ds(ibase + c * SUB, SUB)], buf)

        def fire_g(ib, rb, sem):
            for j in range(SUB):
                pltpu.async_copy(tab_hbm.at[ib.at[j]],
                                 rb.at[pl.ds(j * IDXW, IDXW)], sem)

        def wait_g(ib, rb, sem):
            for j in range(SUB):
                pltpu.make_async_copy(tab_hbm.at[ib.at[j]],
                                      rb.at[pl.ds(j * IDXW, IDXW)], sem).wait()

        def fire_s(rb, c, sem):
            pltpu.async_copy(rb, out_hbm.at[pl.ds(base + c * CHUNK, CHUNK)], sem)

        def wait_s(rb, sem):
            pltpu.make_async_copy(rb, out_hbm.at[pl.ds(base, CHUNK)], sem).wait()

        idxload(i0, 0)
        fire_g(i0, r0, sg0)

        def body(i, carry):
            c0 = 2 * i
            c1 = c0 + 1
            idxload(i1, c1)

            @pl.when(i > 0)
            def _():
                wait_s(r1, ss1)

            fire_g(i1, r1, sg1)
            wait_g(i0, r0, sg0)
            fire_s(r0, c0, ss0)

            @pl.when(i < nhalf - 1)
            def _():
                idxload(i0, c0 + 2)
                wait_s(r0, ss0)
                fire_g(i0, r0, sg0)

            wait_g(i1, r1, sg1)
            fire_s(r1, c1, ss1)
            return carry

        lax.fori_loop(0, nhalf, body, 0)
        wait_s(r0, ss0)
        wait_s(r1, ss1)

    return k


def _head_body(K, MB, PK, g_ref, sup_ref, w1p_ref, b1_ref, w2b_ref, b2_ref,
               o_ref):
    KP = K // PK
    H1 = w1p_ref.shape[0]
    t = lax.dot_general(sup_ref[0], w1p_ref[...], (((1,), (1,)), ((), ())),
                        preferred_element_type=jnp.float32) - b1_ref[...]
    t4 = jnp.concatenate([t] * PK, axis=1)
    g3 = g_ref[0].reshape(MB, KP, PK * H1)
    r = jnp.maximum(g3 - t4[:, None, :], 0.0).reshape(MB * KP, PK * H1)
    h = lax.dot_general(r, w2b_ref[...], (((1,), (0,)), ((), ())),
                        preferred_element_type=jnp.float32)
    OUT = o_ref.shape[2]
    hm = jnp.max(h.reshape(MB, KP, PK * OUT), axis=1)
    o = hm[:, :OUT]
    for q in range(1, PK):
        o = jnp.maximum(o, hm[:, q * OUT:(q + 1) * OUT])
    o_ref[0] = o + b2_ref[...]


def _make_head(B, M, K, H1, OUT, MB, PK):
    nmb = M // MB
    KP = K // PK
    return pl.pallas_call(
        functools.partial(_head_body, K, MB, PK),
        grid=(B, nmb),
        in_specs=[
            pl.BlockSpec((1, MB * KP, PK * H1), lambda b, i: (b, i, 0)),
            pl.BlockSpec((1, MB, 3), lambda b, i: (b, i, 0)),
            pl.BlockSpec((H1, 3), lambda b, i: (0, 0)),
            pl.BlockSpec((1, H1), lambda b, i: (0, 0)),
            pl.BlockSpec((PK * H1, PK * OUT), lambda b, i: (0, 0)),
            pl.BlockSpec((1, OUT), lambda b, i: (0, 0)),
        ],
        out_specs=pl.BlockSpec((1, MB, OUT), lambda b, i: (b, i, 0)),
        out_shape=jax.ShapeDtypeStruct((B, M, OUT), jnp.float32),
        compiler_params=pltpu.CompilerParams(
            dimension_semantics=("parallel", "parallel")),
    )


def kernel(x, pos, support_points, indices, W1, b1, W2, b2):
    B, C, N = x.shape
    _, M, K = indices.shape
    H1 = W1.shape[0]
    OUT = W2.shape[0]
    W1x = W1[:, :C]
    W1p = W1[:, C:]

    z = _make_ztab(B, C, N, H1)(x, pos, W1x, W1p)

    BRb = M * K
    IDXW = 1000
    PK = 128 // H1
    MB = 1000
    KP = K // PK
    supT = support_points.transpose(0, 2, 1)
    W2blk = jnp.kron(jnp.eye(PK, dtype=W2.dtype), W2.T)
    b1r = b1.reshape(1, H1)
    b2r = b2.reshape(1, OUT)

    gather = _make_sc_gather(BRb, H1, IDXW)
    head = _make_head(1, M, K, H1, OUT, MB=MB, PK=PK)
    outs = []
    for b in range(B):
        idxf = indices[b].reshape(BRb // IDXW, IDXW)
        g = gather(z[b], idxf)
        g4 = g.reshape(1, M * K // PK, PK * H1)
        outs.append(head(g4, supT[b:b + 1], W1p, b1r, W2blk, b2r))
    out = jnp.concatenate(outs, axis=0)
    return out.transpose(0, 2, 1)
